# baseline (speedup 1.0000x reference)
import functools

import jax
import jax.numpy as jnp
from jax import lax
from jax.experimental import pallas as pl
from jax.experimental.pallas import tpu as pltpu
from jax.experimental.pallas import tpu_sc as plsc

_B = 4096
_S = 200
_D = 64
_DP = 128
_G = 100
_NC = 2
_NS = 16
_NW = _NC * _NS
_NB = _B // _NW
_L = 16


def _gather_body(idx_hbm, table_hbm, out_hbm, idx_v, rowsa_v, rowsb_v,
                 slab_v, sem):
    wid = lax.axis_index("s") * _NC + lax.axis_index("c")
    b0 = wid * _NB
    i0 = pl.multiple_of(wid * (2 * _NB), 8)
    pltpu.sync_copy(idx_hbm.at[pl.ds(i0, 2 * _NB)], idx_v)

    def step(c, carry):
        ga = pltpu.async_copy(table_hbm.at[idx_v.at[2 * c]], rowsa_v, sem)
        gb = pltpu.async_copy(table_hbm.at[idx_v.at[2 * c + 1]], rowsb_v, sem)
        ga.wait()
        gb.wait()

        def trim(r2, carry2):
            for u in range(2):
                r = r2 * 2 + u
                for q in range(_D // _L):
                    slab_v[r, pl.ds(q * _L, _L)] = (
                        rowsa_v[r, pl.ds(q * _L, _L)])
                    slab_v[_G + r, pl.ds(q * _L, _L)] = (
                        rowsb_v[r, pl.ds(q * _L, _L)])
            return carry2

        lax.fori_loop(0, _G // 2, trim, 0, unroll=2)
        pltpu.sync_copy(slab_v, out_hbm.at[b0 + c])
        return carry

    lax.fori_loop(0, _NB, step, 0)


_mesh = plsc.VectorSubcoreMesh(core_axis_name="c", subcore_axis_name="s")

_gather = functools.partial(
    pl.kernel,
    out_type=jax.ShapeDtypeStruct((_B, _S, _D), jnp.float32),
    mesh=_mesh,
    scratch_types=[
        pltpu.VMEM((2 * _NB, _G), jnp.int32),
        pltpu.VMEM((_G, _DP), jnp.float32),
        pltpu.VMEM((_G, _DP), jnp.float32),
        pltpu.VMEM((_S, _D), jnp.float32),
        pltpu.SemaphoreType.DMA,
    ],
)(_gather_body)


def kernel(gene_indices, table):
    idx2 = gene_indices.reshape(2 * _B, _G)
    table_pad = jnp.pad(table, ((0, 0), (0, _DP - _D)))
    return _gather(idx2, table_pad)


# speedup vs baseline: 2.2265x; 2.2265x over previous
import functools

import jax
import jax.numpy as jnp
from jax import lax
from jax.experimental import pallas as pl
from jax.experimental.pallas import tpu as pltpu
from jax.experimental.pallas import tpu_sc as plsc

_B = 4096
_S = 200
_D = 64
_DP = 128
_NC = 2
_NS = 16
_NW = _NC * _NS
_NB = _B // _NW
_GA = 96
_GB = _S - _GA


def _gather_body(idx_hbm, table_hbm, out_hbm, idx_v, rows_v, sem):
    wid = lax.axis_index("s") * _NC + lax.axis_index("c")
    b0 = wid * _NB
    pltpu.sync_copy(idx_hbm.at[pl.ds(b0, _NB)], idx_v)

    def step(c, carry):
        ga = pltpu.async_copy(
            table_hbm.at[idx_v.at[c, pl.ds(0, _GA)]],
            rows_v.at[pl.ds(0, _GA)], sem)
        gb = pltpu.async_copy(
            table_hbm.at[idx_v.at[c, pl.ds(_GA, _GB)]],
            rows_v.at[pl.ds(_GA, _GB)], sem)
        ga.wait()
        gb.wait()
        pltpu.sync_copy(rows_v, out_hbm.at[b0 + c, :, pl.ds(0, _D)])
        return carry

    lax.fori_loop(0, _NB, step, 0)


_mesh = plsc.VectorSubcoreMesh(core_axis_name="c", subcore_axis_name="s")

_gather = functools.partial(
    pl.kernel,
    out_type=jax.ShapeDtypeStruct((_B, _S, _DP), jnp.float32),
    mesh=_mesh,
    scratch_types=[
        pltpu.VMEM((_NB, _S), jnp.int32),
        pltpu.VMEM((_S, _D), jnp.float32),
        pltpu.SemaphoreType.DMA,
    ],
    compiler_params=pltpu.CompilerParams(use_tc_tiling_on_sc=False),
)(_gather_body)


def kernel(gene_indices, table):
    wide = _gather(gene_indices, table)
    return lax.slice(wide, (0, 0, 0), (_B, _S, _D))


# speedup vs baseline: 2.5701x; 1.1543x over previous
import functools

import jax
import jax.numpy as jnp
from jax import lax
from jax.experimental import pallas as pl
from jax.experimental.pallas import tpu as pltpu
from jax.experimental.pallas import tpu_sc as plsc

_B = 4096
_S = 200
_D = 64
_DP = 128
_NC = 2
_NS = 16
_NW = _NC * _NS
_NB = _B // _NW
_GA = 96
_GB = _S - _GA


def _gather_body(idx_hbm, table_hbm, out_hbm, idx_v, rows0_v, rows1_v,
                 sg0, sg1, sw0, sw1):
    wid = lax.axis_index("s") * _NC + lax.axis_index("c")
    b0 = wid * _NB
    pltpu.sync_copy(idx_hbm.at[pl.ds(b0, _NB)], idx_v)

    def fire_gather(c, buf, sem):
        pltpu.async_copy(table_hbm.at[idx_v.at[c, pl.ds(0, _GA)]],
                         buf.at[pl.ds(0, _GA)], sem)
        pltpu.async_copy(table_hbm.at[idx_v.at[c, pl.ds(_GA, _GB)]],
                         buf.at[pl.ds(_GA, _GB)], sem)

    def wait_gather(c, buf, sem):
        pltpu.make_async_copy(table_hbm.at[idx_v.at[c, pl.ds(0, _GA)]],
                              buf.at[pl.ds(0, _GA)], sem).wait()
        pltpu.make_async_copy(table_hbm.at[idx_v.at[c, pl.ds(_GA, _GB)]],
                              buf.at[pl.ds(_GA, _GB)], sem).wait()

    def fire_write(c, buf, sem):
        pltpu.async_copy(buf, out_hbm.at[b0 + c, :, pl.ds(0, _D)], sem)

    def wait_write(c, buf, sem):
        pltpu.make_async_copy(buf, out_hbm.at[b0 + c, :, pl.ds(0, _D)],
                              sem).wait()

    fire_gather(0, rows0_v, sg0)
    fire_gather(1, rows1_v, sg1)

    def step(i, carry):
        c0 = i * 2
        wait_gather(c0, rows0_v, sg0)
        fire_write(c0, rows0_v, sw0)
        wait_gather(c0 + 1, rows1_v, sg1)
        fire_write(c0 + 1, rows1_v, sw1)

        @pl.when(i < _NB // 2 - 1)
        def _refill():
            wait_write(c0, rows0_v, sw0)
            fire_gather(c0 + 2, rows0_v, sg0)
            wait_write(c0 + 1, rows1_v, sw1)
            fire_gather(c0 + 3, rows1_v, sg1)

        return carry

    lax.fori_loop(0, _NB // 2, step, 0)
    wait_write(_NB - 2, rows0_v, sw0)
    wait_write(_NB - 1, rows1_v, sw1)


_mesh = plsc.VectorSubcoreMesh(core_axis_name="c", subcore_axis_name="s")

_gather = functools.partial(
    pl.kernel,
    out_type=jax.ShapeDtypeStruct((_B, _S, _DP), jnp.float32),
    mesh=_mesh,
    scratch_types=[
        pltpu.VMEM((_NB, _S), jnp.int32),
        pltpu.VMEM((_S, _D), jnp.float32),
        pltpu.VMEM((_S, _D), jnp.float32),
        pltpu.SemaphoreType.DMA,
        pltpu.SemaphoreType.DMA,
        pltpu.SemaphoreType.DMA,
        pltpu.SemaphoreType.DMA,
    ],
    compiler_params=pltpu.CompilerParams(use_tc_tiling_on_sc=False),
)(_gather_body)


def kernel(gene_indices, table):
    wide = _gather(gene_indices, table)
    return lax.slice(wide, (0, 0, 0), (_B, _S, _D))


# speedup vs baseline: 2.6843x; 1.0444x over previous
import functools

import jax
import jax.numpy as jnp
from jax import lax
from jax.experimental import pallas as pl
from jax.experimental.pallas import tpu as pltpu
from jax.experimental.pallas import tpu_sc as plsc

_B = 4096
_S = 200
_D = 64
_DP = 128
_NC = 2
_NS = 16
_NW = _NC * _NS
_NB = _B // _NW
_GA = 96
_GB = _S - _GA


def _gather_body(idx_hbm, table_hbm, out_hbm, idx_v,
                 rows0_v, rows1_v, rows2_v, rows3_v,
                 sg0, sg1, sg2, sg3, sw0, sw1, sw2, sw3):
    wid = lax.axis_index("s") * _NC + lax.axis_index("c")
    b0 = wid * _NB
    pltpu.sync_copy(idx_hbm.at[pl.ds(b0, _NB)], idx_v)

    def fire_gather(c, buf, sem):
        pltpu.async_copy(table_hbm.at[idx_v.at[c, pl.ds(0, _GA)]],
                         buf.at[pl.ds(0, _GA)], sem)
        pltpu.async_copy(table_hbm.at[idx_v.at[c, pl.ds(_GA, _GB)]],
                         buf.at[pl.ds(_GA, _GB)], sem)

    def wait_gather(c, buf, sem):
        pltpu.make_async_copy(table_hbm.at[idx_v.at[c, pl.ds(0, _GA)]],
                              buf.at[pl.ds(0, _GA)], sem).wait()
        pltpu.make_async_copy(table_hbm.at[idx_v.at[c, pl.ds(_GA, _GB)]],
                              buf.at[pl.ds(_GA, _GB)], sem).wait()

    def fire_write(c, buf, sem):
        pltpu.async_copy(buf, out_hbm.at[b0 + c, :, pl.ds(0, _D)], sem)

    def wait_write(c, buf, sem):
        pltpu.make_async_copy(buf, out_hbm.at[b0 + c, :, pl.ds(0, _D)],
                              sem).wait()

    bufs = (rows0_v, rows1_v, rows2_v, rows3_v)
    sgs = (sg0, sg1, sg2, sg3)
    sws = (sw0, sw1, sw2, sw3)
    for j in range(4):
        fire_gather(j, bufs[j], sgs[j])

    def step(i, carry):
        c0 = i * 4
        for j in range(4):
            wait_gather(c0 + j, bufs[j], sgs[j])
            fire_write(c0 + j, bufs[j], sws[j])

        @pl.when(i < _NB // 4 - 1)
        def _refill():
            for j in range(4):
                wait_write(c0 + j, bufs[j], sws[j])
                fire_gather(c0 + 4 + j, bufs[j], sgs[j])

        return carry

    lax.fori_loop(0, _NB // 4, step, 0)
    for j in range(4):
        wait_write(_NB - 4 + j, bufs[j], sws[j])


_mesh = plsc.VectorSubcoreMesh(core_axis_name="c", subcore_axis_name="s")

_gather = functools.partial(
    pl.kernel,
    out_type=jax.ShapeDtypeStruct((_B, _S, _DP), jnp.float32),
    mesh=_mesh,
    scratch_types=(
        [pltpu.VMEM((_NB, _S), jnp.int32)]
        + [pltpu.VMEM((_S, _D), jnp.float32)] * 4
        + [pltpu.SemaphoreType.DMA] * 8
    ),
    compiler_params=pltpu.CompilerParams(use_tc_tiling_on_sc=False),
)(_gather_body)


def kernel(gene_indices, table):
    wide = _gather(gene_indices, table)
    return lax.slice(wide, (0, 0, 0), (_B, _S, _D))
